# baseline (speedup 1.0000x reference)
import functools

import jax
import jax.numpy as jnp
from jax import lax
from jax.experimental import pallas as pl
from jax.experimental.pallas import tpu as pltpu
from jax.experimental.pallas import tpu_sc as plsc


def _sc_gather(table, idx_flat, nch, ch):
    n_total = idx_flat.shape[0]
    d = table.shape[1]
    n_per_w = nch * ch
    mesh = plsc.VectorSubcoreMesh(core_axis_name="c", subcore_axis_name="s")
    info = plsc.get_sparse_core_info()
    nc = info.num_cores

    @functools.partial(
        pl.kernel,
        mesh=mesh,
        out_type=jax.ShapeDtypeStruct((n_total, d), jnp.float32),
        scratch_types=[
            pltpu.VMEM((n_per_w,), jnp.int32),
            pltpu.VMEM((2, ch, d), jnp.float32),
            pltpu.SemaphoreType.DMA,
            pltpu.SemaphoreType.DMA,
        ],
        compiler_params=pltpu.CompilerParams(use_tc_tiling_on_sc=False),
    )
    def k(table_hbm, idx_hbm, out_hbm, idx_v, rows_v, sem0, sem1):
        wid = lax.axis_index("s") * nc + lax.axis_index("c")
        base = wid * n_per_w
        pltpu.sync_copy(idx_hbm.at[pl.ds(base, n_per_w)], idx_v)
        sems = (sem0, sem1)
        cps = [None, None]
        for j in range(nch):
            b = j % 2
            cps[b] = pltpu.async_copy(
                table_hbm.at[idx_v.at[pl.ds(j * ch, ch)]],
                rows_v.at[b],
                sems[b],
            )
            if j >= 1:
                bp = (j - 1) % 2
                cps[bp].wait()
                pltpu.sync_copy(
                    rows_v.at[bp],
                    out_hbm.at[pl.ds(base + (j - 1) * ch, ch)],
                )
        bl_ = (nch - 1) % 2
        cps[bl_].wait()
        pltpu.sync_copy(
            rows_v.at[bl_],
            out_hbm.at[pl.ds(base + (nch - 1) * ch, ch)],
        )

    return k(table, idx_flat)


def _attn_body(emb_ref, pos_ref, wk_ref, wq_ref, wv_ref, wl_ref, bl_ref,
               mask_ref, out_ref, *, scale, bb, tp, t_out):
    e = emb_ref[...] + pos_ref[...]
    q = jnp.dot(e, wq_ref[...], preferred_element_type=jnp.float32)
    k = jnp.dot(e, wk_ref[...], preferred_element_type=jnp.float32)
    v = jnp.dot(e, wv_ref[...], preferred_element_type=jnp.float32)
    wei = lax.dot_general(q, k, (((1,), (1,)), ((), ())),
                          preferred_element_type=jnp.float32)
    wei = wei * scale + mask_ref[...]
    m = jnp.max(wei, axis=1, keepdims=True)
    p = jnp.exp(wei - m)
    s = jnp.sum(p, axis=1, keepdims=True)
    o = jnp.dot(p, v, preferred_element_type=jnp.float32) / s
    logits = jnp.dot(o, wl_ref[...],
                     preferred_element_type=jnp.float32) + bl_ref[...]
    vv = logits.shape[1]
    for b in range(bb):
        out_ref[b] = lax.slice(logits, (b * tp, 0), (b * tp + t_out, vv))


def _tc_attn_logits(emb2d, pos_tiled, Wk, Wq, Wv, Wl, bl2d, mask_add,
                    bb, tp, t_out, n_batch):
    D = Wl.shape[0]
    V = Wl.shape[1]
    R = bb * tp
    grid = n_batch // bb
    scale = float(D) ** -0.5
    return pl.pallas_call(
        functools.partial(_attn_body, scale=scale, bb=bb, tp=tp, t_out=t_out),
        grid=(grid,),
        in_specs=[
            pl.BlockSpec((R, D), lambda i: (i, 0)),
            pl.BlockSpec((R, D), lambda i: (0, 0)),
            pl.BlockSpec((D, D), lambda i: (0, 0)),
            pl.BlockSpec((D, D), lambda i: (0, 0)),
            pl.BlockSpec((D, D), lambda i: (0, 0)),
            pl.BlockSpec((D, V), lambda i: (0, 0)),
            pl.BlockSpec((1, V), lambda i: (0, 0)),
            pl.BlockSpec((R, R), lambda i: (0, 0)),
        ],
        out_specs=pl.BlockSpec((bb, t_out, V), lambda i: (i, 0, 0)),
        out_shape=jax.ShapeDtypeStruct((n_batch, t_out, V), jnp.float32),
        compiler_params=pltpu.CompilerParams(
            dimension_semantics=("parallel",),
        ),
    )(emb2d, pos_tiled, Wk, Wq, Wv, Wl, bl2d, mask_add)


def kernel(x, tok_table, pos_table, Wk, Wq, Wv, Wl, bl):
    B, T = x.shape
    V, D = tok_table.shape
    TP = 56
    N = B * TP

    BB = 8
    R = BB * TP

    info = plsc.get_sparse_core_info()
    NW = info.num_cores * info.num_subcores
    n_per_w = N // NW
    CH = 112
    NCH = n_per_w // CH
    idx_flat = jnp.pad(x.astype(jnp.int32), ((0, 0), (0, TP - T))).reshape(N)
    emb2d = _sc_gather(tok_table, idx_flat, NCH, CH)

    pos_pad = jnp.pad(pos_table, ((0, TP - T), (0, 0)))
    pos_tiled = jnp.tile(pos_pad, (BB, 1))
    r = jnp.arange(R)
    bidx, t = r // TP, r % TP
    causal = ((bidx[:, None] == bidx[None, :])
              & (t[:, None] >= t[None, :])
              & (t[None, :] < T))
    mask_add = jnp.where(causal, 0.0, -1e30).astype(jnp.float32)
    return _tc_attn_logits(emb2d, pos_tiled, Wk, Wq, Wv, Wl,
                           bl.reshape(1, V), mask_add, BB, TP, T, B)
